# baseline (speedup 1.0000x reference)
import functools

import jax
import jax.numpy as jnp
import numpy as np
from jax.experimental import pallas as pl
from jax.experimental.pallas import tpu as pltpu

_N = 10000
_NPAD = 10240
_E = 320000
_CHUNK = 2000
_NSTEPS = _E // _CHUNK
_ROWBLK = 512
_DELTA = 2.5
_BN_EPS = 1e-5


def _ab_body(h_ref, w1_ref, w2_ref, mb_ref, a_ref, b_ref):
    h = h_ref[...]
    a_ref[...] = jnp.dot(h, w1_ref[...], preferred_element_type=jnp.float32)
    b_ref[...] = (
        jnp.dot(h, w2_ref[...], preferred_element_type=jnp.float32) + mb_ref[...]
    )


def _scatter_body(src_ref, dst_ref, a_ref, acc_ref):
    step = pl.program_id(0)

    @pl.when(step == 0)
    def _init():
        npad = acc_ref.shape[0]
        acc_ref[:, pl.ds(0, 128)] = jnp.zeros((npad, 128), jnp.float32)
        acc_ref[:, pl.ds(128, 128)] = jnp.full((npad, 128), jnp.inf, jnp.float32)
        acc_ref[:, pl.ds(256, 128)] = jnp.full((npad, 128), -jnp.inf, jnp.float32)
        acc_ref[:, pl.ds(384, 128)] = jnp.zeros((npad, 128), jnp.float32)

    one = jnp.ones((1, 128), jnp.float32)

    def body(i, carry):
        s = src_ref[0, 0, i]
        d = dst_ref[0, 0, i]
        row = a_ref[pl.ds(s, 1), :]
        cur = acc_ref[pl.ds(d, 1), :]
        acc_ref[pl.ds(d, 1), :] = jnp.concatenate(
            [
                cur[:, 0:128] + row,
                jnp.minimum(cur[:, 128:256], row),
                jnp.maximum(cur[:, 256:384], row),
                cur[:, 384:512] + one,
            ],
            axis=1,
        )
        return carry

    jax.lax.fori_loop(0, _CHUNK, body, 0)


def _post_body(
    h_ref, acc_ref, b_ref,
    uw_ref, ub_ref, bng_ref, bnb_ref, mw_ref, mbb_ref, o_ref,
    *, residual, relu,
):
    deg = acc_ref[:, pl.ds(384, 128)]
    has = deg > 0.0
    b = b_ref[...]
    mean = jnp.where(has, acc_ref[:, pl.ds(0, 128)] / jnp.maximum(deg, 1.0) + b, 0.0)
    mn = jnp.where(has, acc_ref[:, pl.ds(128, 128)] + b, 0.0)
    mx = jnp.where(has, acc_ref[:, pl.ds(256, 128)] + b, 0.0)
    logd = jnp.log(deg + 1.0)
    amp = jnp.where(has, logd / _DELTA, 0.0)
    att = jnp.where(has, _DELTA / jnp.maximum(logd, 1e-12), 0.0)
    h = h_ref[...]
    hcat = jnp.concatenate(
        [h, mean, mn, mx, mean * amp, mn * amp, mx * amp,
         mean * att, mn * att, mx * att],
        axis=1,
    )
    u = jnp.dot(hcat, uw_ref[...], preferred_element_type=jnp.float32) + ub_ref[...]
    u = u * (float(1.0 / np.sqrt(1.0 + _BN_EPS))) * bng_ref[...] + bnb_ref[...]
    u = jnp.dot(u, mw_ref[...], preferred_element_type=jnp.float32) + mbb_ref[...]
    u = jnp.where(u > 0, u, 0.01 * u)
    if residual:
        u = u + h
    if relu:
        u = jnp.maximum(u, 0.0)
    o_ref[...] = u


def _pad_cols(a, width=128):
    if a.shape[-1] == width:
        return a
    pad = [(0, 0)] * (a.ndim - 1) + [(0, width - a.shape[-1])]
    return jnp.pad(a, pad)


_full_spec = pl.BlockSpec((_NPAD, 128), lambda i: (0, 0))


def _layer(h, src2, dst2, layer, residual, relu):
    M_w, M_b, U_w, U_b, bn_g, bn_b, mix_w, mix_b = layer
    di = M_w.shape[0]
    w1 = M_w[:, :di].T
    w2 = M_w[:, di:].T
    mb = M_b.reshape(1, di)
    uw = _pad_cols(U_w.T)
    ub = _pad_cols(U_b.reshape(1, -1))
    bng = _pad_cols(bn_g.reshape(1, -1))
    bnb = _pad_cols(bn_b.reshape(1, -1))
    do = mix_w.shape[0]
    mw = _pad_cols(jnp.pad(mix_w.T, ((0, 128 - do), (0, 0))))
    mbb = _pad_cols(mix_b.reshape(1, -1))

    nblk = _NPAD // _ROWBLK
    wspec = pl.BlockSpec((di, di), lambda i: (0, 0))
    vspec = pl.BlockSpec((1, di), lambda i: (0, 0))
    a, b = pl.pallas_call(
        _ab_body,
        grid=(nblk,),
        in_specs=[
            pl.BlockSpec((_ROWBLK, di), lambda i: (i, 0)),
            wspec, wspec, vspec,
        ],
        out_specs=[
            pl.BlockSpec((_ROWBLK, di), lambda i: (i, 0)),
            pl.BlockSpec((_ROWBLK, di), lambda i: (i, 0)),
        ],
        out_shape=[
            jax.ShapeDtypeStruct((_NPAD, di), jnp.float32),
            jax.ShapeDtypeStruct((_NPAD, di), jnp.float32),
        ],
    )(h, w1, w2, mb)

    idx_spec = pl.BlockSpec(
        (1, 1, _CHUNK), lambda i: (i, 0, 0), memory_space=pltpu.SMEM
    )
    acc = pl.pallas_call(
        _scatter_body,
        grid=(_NSTEPS,),
        in_specs=[idx_spec, idx_spec, _full_spec],
        out_specs=pl.BlockSpec((_NPAD, 512), lambda i: (0, 0)),
        out_shape=jax.ShapeDtypeStruct((_NPAD, 512), jnp.float32),
    )(src2, dst2, a)

    rspec = pl.BlockSpec((_ROWBLK, 128), lambda i: (i, 0))
    accspec = pl.BlockSpec((_ROWBLK, 512), lambda i: (i, 0))
    uwspec = pl.BlockSpec(uw.shape, lambda i: (0, 0))
    cvec = pl.BlockSpec((1, 128), lambda i: (0, 0))
    mwspec = pl.BlockSpec((128, 128), lambda i: (0, 0))
    out = pl.pallas_call(
        functools.partial(_post_body, residual=residual, relu=relu),
        grid=(nblk,),
        in_specs=[rspec, accspec, rspec,
                  uwspec, cvec, cvec, cvec, mwspec, cvec],
        out_specs=rspec,
        out_shape=jax.ShapeDtypeStruct((_NPAD, 128), jnp.float32),
    )(h, acc, b, uw, ub, bng, bnb, mw, mbb)
    return out


def kernel(x, edge_index, params):
    x = jnp.pad(x, ((0, _NPAD - _N), (0, 0)))
    src2 = edge_index[0].reshape(_NSTEPS, 1, _CHUNK)
    dst2 = edge_index[1].reshape(_NSTEPS, 1, _CHUNK)
    residuals = [True, True, False]
    h = x
    for i, layer in enumerate(params):
        h = _layer(h, src2, dst2, layer, residuals[i], relu=(i != len(params) - 1))
    return h[:_N, :64]


# speedup vs baseline: 1.3989x; 1.3989x over previous
import functools

import jax
import jax.numpy as jnp
import numpy as np
from jax.experimental import pallas as pl
from jax.experimental.pallas import tpu as pltpu

_N = 10000
_NPAD = 10240
_E = 320000
_CHUNK = 2000
_NSTEPS = _E // _CHUNK
_ROWBLK = 512
_DELTA = 2.5
_BN_EPS = 1e-5


def _ab_body(h_ref, w1_ref, w2_ref, mb_ref, a_ref, b_ref):
    h = h_ref[...]
    a_ref[...] = jnp.dot(h, w1_ref[...], preferred_element_type=jnp.float32)
    b_ref[...] = (
        jnp.dot(h, w2_ref[...], preferred_element_type=jnp.float32) + mb_ref[...]
    )


def _scatter_body(src_ref, dst_ref, a_ref, s_ref, mn_ref, mx_ref, deg_ref):
    step = pl.program_id(0)

    @pl.when(step == 0)
    def _init():
        s_ref[...] = jnp.zeros_like(s_ref)
        mn_ref[...] = jnp.full_like(mn_ref, jnp.inf)
        mx_ref[...] = jnp.full_like(mx_ref, -jnp.inf)
        deg_ref[...] = jnp.zeros_like(deg_ref)

    one = jnp.ones((1, 128), jnp.float32)

    def body(i, carry):
        s = src_ref[0, 0, i]
        d = dst_ref[0, 0, i]
        row = a_ref[pl.ds(s, 1), :]
        s_ref[pl.ds(d, 1), :] += row
        mn_ref[pl.ds(d, 1), :] = jnp.minimum(mn_ref[pl.ds(d, 1), :], row)
        mx_ref[pl.ds(d, 1), :] = jnp.maximum(mx_ref[pl.ds(d, 1), :], row)
        deg_ref[pl.ds(d, 1), :] += one
        return carry

    jax.lax.fori_loop(0, _CHUNK, body, 0)


def _scatter_body_nodeg(src_ref, dst_ref, a_ref, s_ref, mn_ref, mx_ref):
    step = pl.program_id(0)

    @pl.when(step == 0)
    def _init():
        s_ref[...] = jnp.zeros_like(s_ref)
        mn_ref[...] = jnp.full_like(mn_ref, jnp.inf)
        mx_ref[...] = jnp.full_like(mx_ref, -jnp.inf)

    def body(i, carry):
        s = src_ref[0, 0, i]
        d = dst_ref[0, 0, i]
        row = a_ref[pl.ds(s, 1), :]
        s_ref[pl.ds(d, 1), :] += row
        mn_ref[pl.ds(d, 1), :] = jnp.minimum(mn_ref[pl.ds(d, 1), :], row)
        mx_ref[pl.ds(d, 1), :] = jnp.maximum(mx_ref[pl.ds(d, 1), :], row)
        return carry

    jax.lax.fori_loop(0, _CHUNK, body, 0)


def _post_body(
    h_ref, s_ref, mn_ref, mx_ref, deg_ref, b_ref,
    uw_ref, ub_ref, bng_ref, bnb_ref, mw_ref, mbb_ref, o_ref,
    *, residual, relu,
):
    deg = deg_ref[...]
    has = deg > 0.0
    b = b_ref[...]
    mean = jnp.where(has, s_ref[...] / jnp.maximum(deg, 1.0) + b, 0.0)
    mn = jnp.where(has, mn_ref[...] + b, 0.0)
    mx = jnp.where(has, mx_ref[...] + b, 0.0)
    logd = jnp.log(deg + 1.0)
    amp = jnp.where(has, logd / _DELTA, 0.0)
    att = jnp.where(has, _DELTA / jnp.maximum(logd, 1e-12), 0.0)
    h = h_ref[...]
    hcat = jnp.concatenate(
        [h, mean, mn, mx, mean * amp, mn * amp, mx * amp,
         mean * att, mn * att, mx * att],
        axis=1,
    )
    u = jnp.dot(hcat, uw_ref[...], preferred_element_type=jnp.float32) + ub_ref[...]
    u = u * (float(1.0 / np.sqrt(1.0 + _BN_EPS))) * bng_ref[...] + bnb_ref[...]
    u = jnp.dot(u, mw_ref[...], preferred_element_type=jnp.float32) + mbb_ref[...]
    u = jnp.where(u > 0, u, 0.01 * u)
    if residual:
        u = u + h
    if relu:
        u = jnp.maximum(u, 0.0)
    o_ref[...] = u


def _pad_cols(a, width=128):
    if a.shape[-1] == width:
        return a
    pad = [(0, 0)] * (a.ndim - 1) + [(0, width - a.shape[-1])]
    return jnp.pad(a, pad)


_full_spec = pl.BlockSpec((_NPAD, 128), lambda i: (0, 0))


def _layer(h, src2, dst2, layer, residual, relu, deg=None):
    M_w, M_b, U_w, U_b, bn_g, bn_b, mix_w, mix_b = layer
    di = M_w.shape[0]
    w1 = M_w[:, :di].T
    w2 = M_w[:, di:].T
    mb = M_b.reshape(1, di)
    uw = _pad_cols(U_w.T)
    ub = _pad_cols(U_b.reshape(1, -1))
    bng = _pad_cols(bn_g.reshape(1, -1))
    bnb = _pad_cols(bn_b.reshape(1, -1))
    do = mix_w.shape[0]
    mw = _pad_cols(jnp.pad(mix_w.T, ((0, 128 - do), (0, 0))))
    mbb = _pad_cols(mix_b.reshape(1, -1))

    nblk = _NPAD // _ROWBLK
    wspec = pl.BlockSpec((di, di), lambda i: (0, 0))
    vspec = pl.BlockSpec((1, di), lambda i: (0, 0))
    a, b = pl.pallas_call(
        _ab_body,
        grid=(nblk,),
        in_specs=[
            pl.BlockSpec((_ROWBLK, di), lambda i: (i, 0)),
            wspec, wspec, vspec,
        ],
        out_specs=[
            pl.BlockSpec((_ROWBLK, di), lambda i: (i, 0)),
            pl.BlockSpec((_ROWBLK, di), lambda i: (i, 0)),
        ],
        out_shape=[
            jax.ShapeDtypeStruct((_NPAD, di), jnp.float32),
            jax.ShapeDtypeStruct((_NPAD, di), jnp.float32),
        ],
    )(h, w1, w2, mb)

    idx_spec = pl.BlockSpec(
        (1, 1, _CHUNK), lambda i: (i, 0, 0), memory_space=pltpu.SMEM
    )
    if deg is None:
        s, mn, mx, deg = pl.pallas_call(
            _scatter_body,
            grid=(_NSTEPS,),
            in_specs=[idx_spec, idx_spec, _full_spec],
            out_specs=[_full_spec] * 4,
            out_shape=[jax.ShapeDtypeStruct((_NPAD, 128), jnp.float32)] * 4,
        )(src2, dst2, a)
    else:
        s, mn, mx = pl.pallas_call(
            _scatter_body_nodeg,
            grid=(_NSTEPS,),
            in_specs=[idx_spec, idx_spec, _full_spec],
            out_specs=[_full_spec] * 3,
            out_shape=[jax.ShapeDtypeStruct((_NPAD, 128), jnp.float32)] * 3,
        )(src2, dst2, a)

    rspec = pl.BlockSpec((_ROWBLK, 128), lambda i: (i, 0))
    uwspec = pl.BlockSpec(uw.shape, lambda i: (0, 0))
    cvec = pl.BlockSpec((1, 128), lambda i: (0, 0))
    mwspec = pl.BlockSpec((128, 128), lambda i: (0, 0))
    out = pl.pallas_call(
        functools.partial(_post_body, residual=residual, relu=relu),
        grid=(nblk,),
        in_specs=[rspec, rspec, rspec, rspec, rspec, rspec,
                  uwspec, cvec, cvec, cvec, mwspec, cvec],
        out_specs=rspec,
        out_shape=jax.ShapeDtypeStruct((_NPAD, 128), jnp.float32),
    )(h, s, mn, mx, deg, b, uw, ub, bng, bnb, mw, mbb)
    return out, deg


def kernel(x, edge_index, params):
    x = jnp.pad(x, ((0, _NPAD - _N), (0, 0)))
    src2 = edge_index[0].reshape(_NSTEPS, 1, _CHUNK)
    dst2 = edge_index[1].reshape(_NSTEPS, 1, _CHUNK)
    residuals = [True, True, False]
    h = x
    deg = None
    for i, layer in enumerate(params):
        h, deg = _layer(
            h, src2, dst2, layer, residuals[i],
            relu=(i != len(params) - 1), deg=deg,
        )
    return h[:_N, :64]
